# Initial kernel scaffold; baseline (speedup 1.0000x reference)
import jax
import jax.numpy as jnp
from jax.experimental import pallas as pl


def kernel(iword, owords, Wi, Wo):
    raise NotImplementedError("write your pallas kernel here")


# speedup vs baseline: 5.2371x; 5.2371x over previous
import functools

import jax
import jax.numpy as jnp
import numpy as np
from jax import lax
from jax.experimental import pallas as pl
from jax.experimental.pallas import tpu as pltpu
from jax.experimental.pallas import tpu_sc as plsc

_N_NEGS = 20
_PAD_COLS = 432
_NCORES = 2
_NSUB = 16
_NW = _NCORES * _NSUB


def _neg_indices(b, n):
    nkey = jax.random.key(42)
    return jax.random.randint(nkey, (b, n), 0, 100000)


def _sc_scores_body(iword_hbm, idx_hbm, wi_hbm, wo_hbm, out_hbm,
                    iwd_v, ivec_v, idx_all_v, rows_v, score_v, sem):
    bpw = ivec_v.shape[0]
    c = lax.axis_index("c")
    s = lax.axis_index("s")
    wid = s * _NCORES + c
    base = wid * bpw

    pltpu.sync_copy(iword_hbm.at[pl.ds(base, bpw)], iwd_v)
    pltpu.sync_copy(idx_hbm.at[pl.ds(base, bpw)], idx_all_v)
    half = bpw // 2
    cp0 = pltpu.async_copy(wi_hbm.at[iwd_v.at[pl.ds(0, half)]],
                           ivec_v.at[pl.ds(0, half)], sem)
    cp1 = pltpu.async_copy(wi_hbm.at[iwd_v.at[pl.ds(half, half)]],
                           ivec_v.at[pl.ds(half, half)], sem)
    cp0.wait()
    cp1.wait()

    lane = lax.iota(jnp.int32, 16)

    def b_body(bl, carry):
        cps = [pltpu.async_copy(wo_hbm.at[idx_all_v.at[bl, j]],
                                rows_v.at[pl.ds(j * 108, 108)], sem)
               for j in range(4)]
        for cp in cps:
            cp.wait()

        ivqs = [ivec_v[bl, pl.ds(q * 16, 16)] for q in range(4)]

        def g_body(g, carry2):
            acc = jnp.zeros((16,), jnp.float32)
            for rr in range(16):
                r = g * 16 + rr
                prod = rows_v[r, pl.ds(0, 16)] * ivqs[0]
                for q in range(1, 4):
                    prod = prod + rows_v[r, pl.ds(q * 16, 16)] * ivqs[q]
                sc = jnp.sum(prod)
                acc = jnp.where(lane == rr, acc + sc, acc)
            score_v[pl.ds(g * 16, 16)] = acc
            return carry2

        lax.fori_loop(0, _PAD_COLS // 16, g_body, 0, unroll=False)
        pltpu.sync_copy(score_v, out_hbm.at[base + bl])
        return carry

    lax.fori_loop(0, bpw, b_body, 0, unroll=False)


def _sc_scores(iword, allidx3, wi, wo):
    b = iword.shape[0]
    bpw = b // _NW
    mesh = plsc.VectorSubcoreMesh(core_axis_name="c", subcore_axis_name="s")
    return pl.kernel(
        _sc_scores_body,
        out_type=jax.ShapeDtypeStruct((b, _PAD_COLS), jnp.float32),
        mesh=mesh,
        compiler_params=pltpu.CompilerParams(
            needs_layout_passes=False, use_tc_tiling_on_sc=False),
        scratch_types=[
            pltpu.VMEM((bpw,), jnp.int32),
            pltpu.VMEM((bpw, 64), jnp.float32),
            pltpu.VMEM((bpw, 4, 108), jnp.int32),
            pltpu.VMEM((_PAD_COLS, 64), jnp.float32),
            pltpu.VMEM((_PAD_COLS,), jnp.float32),
            pltpu.SemaphoreType.DMA,
        ],
    )(iword, allidx3, wi, wo)


def _tc_loss_body(c, scores_ref, out_ref):
    x = scores_ref[...]
    rows, cols = x.shape
    col = lax.broadcasted_iota(jnp.int32, (rows, cols), 1)
    sign = jnp.where(col < c, 1.0, -1.0).astype(jnp.float32)
    z = x * sign
    ls = jnp.minimum(z, 0.0) - jnp.log1p(jnp.exp(-jnp.abs(z)))
    ls = jnp.where(col < c * (1 + _N_NEGS), ls, 0.0)
    out_ref[0, 0] = -jnp.sum(ls) / (rows * c)


def _tc_loss(scores, c):
    return pl.pallas_call(
        functools.partial(_tc_loss_body, c),
        out_shape=jax.ShapeDtypeStruct((1, 1), jnp.float32),
        out_specs=pl.BlockSpec(memory_space=pltpu.SMEM),
    )(scores)


def kernel(iword, owords, Wi, Wo):
    b = iword.shape[0]
    c = owords.shape[1]
    nwords = _neg_indices(b, c * _N_NEGS).astype(jnp.int32)
    pad = jnp.zeros((b, _PAD_COLS - c * (1 + _N_NEGS)), jnp.int32)
    allidx = jnp.concatenate(
        [owords.astype(jnp.int32), nwords, pad], axis=1)
    allidx3 = allidx.reshape(b, 4, 108)
    scores = _sc_scores(iword.astype(jnp.int32), allidx3, Wi, Wo)
    loss = _tc_loss(scores, c)
    return loss.reshape(())


# speedup vs baseline: 8.4505x; 1.6136x over previous
import functools

import jax
import jax.numpy as jnp
import numpy as np
from jax import lax
from jax.experimental import pallas as pl
from jax.experimental.pallas import tpu as pltpu
from jax.experimental.pallas import tpu_sc as plsc

_N_NEGS = 20
_PAD_COLS = 432
_NCORES = 2
_NSUB = 16
_NW = _NCORES * _NSUB


def _neg_indices(b, n):
    nkey = jax.random.key(42)
    return jax.random.randint(nkey, (b, n), 0, 100000)


def _sc_scores_body(iword_hbm, idx_hbm, wi_hbm, wo_hbm, out_hbm,
                    iwd_v, ivec_v, idx_all_v, rows_v, score_v, sem):
    bpw = ivec_v.shape[0]
    c = lax.axis_index("c")
    s = lax.axis_index("s")
    wid = s * _NCORES + c
    base = wid * bpw

    pltpu.sync_copy(iword_hbm.at[pl.ds(base, bpw)], iwd_v)
    pltpu.sync_copy(idx_hbm.at[pl.ds(base, bpw)], idx_all_v)
    half = bpw // 2
    cp0 = pltpu.async_copy(wi_hbm.at[iwd_v.at[pl.ds(0, half)]],
                           ivec_v.at[pl.ds(0, half)], sem)
    cp1 = pltpu.async_copy(wi_hbm.at[iwd_v.at[pl.ds(half, half)]],
                           ivec_v.at[pl.ds(half, half)], sem)
    cp0.wait()
    cp1.wait()

    lane = lax.iota(jnp.int32, 16)

    def issue(bl, k):
        for j in range(4):
            pltpu.async_copy(
                wo_hbm.at[idx_all_v.at[bl, j]],
                rows_v.at[pl.ds(k * _PAD_COLS + j * 108, 108)], sem)

    def drain(bl, k):
        for j in range(4):
            pltpu.make_async_copy(
                wo_hbm.at[idx_all_v.at[bl, j]],
                rows_v.at[pl.ds(k * _PAD_COLS + j * 108, 108)], sem).wait()

    def pair_sum(p):
        u = plsc.bitcast(p, jnp.int32)
        hi = plsc.bitcast(u & jnp.int32(-65536), jnp.float32)
        lo = plsc.bitcast(u << 16, jnp.float32)
        return hi + lo

    def compute(bl, k):
        iv0 = ivec_v[bl, pl.ds(0, 32)]
        iv1 = ivec_v[bl, pl.ds(32, 32)]

        def g_body(g, carry2):
            acc = jnp.zeros((16,), jnp.float32)
            for rr in range(16):
                r = k * _PAD_COLS + g * 16 + rr
                p0 = rows_v[r, pl.ds(0, 32)] * iv0
                p1 = rows_v[r, pl.ds(32, 32)] * iv1
                sc = jnp.sum(pair_sum(p0) + pair_sum(p1))
                acc = jnp.where(lane == rr, acc + sc, acc)
            score_v[pl.ds(g * 16, 16)] = acc
            return carry2

        lax.fori_loop(0, _PAD_COLS // 16, g_body, 0, unroll=False)
        pltpu.sync_copy(score_v, out_hbm.at[base + bl])

    issue(0, 0)

    def pair_body(bp, carry):
        b0 = 2 * bp
        b1 = b0 + 1
        issue(b1, 1)
        drain(b0, 0)
        compute(b0, 0)

        @pl.when(b0 + 2 < bpw)
        def _():
            issue(b0 + 2, 0)

        drain(b1, 1)
        compute(b1, 1)
        return carry

    lax.fori_loop(0, bpw // 2, pair_body, 0, unroll=False)


def _sc_scores(iword, allidx3, wi, wo):
    b = iword.shape[0]
    bpw = b // _NW
    mesh = plsc.VectorSubcoreMesh(core_axis_name="c", subcore_axis_name="s")
    return pl.kernel(
        _sc_scores_body,
        out_type=jax.ShapeDtypeStruct((b, _PAD_COLS), jnp.float32),
        mesh=mesh,
        compiler_params=pltpu.CompilerParams(
            needs_layout_passes=False, use_tc_tiling_on_sc=False),
        scratch_types=[
            pltpu.VMEM((bpw,), jnp.int32),
            pltpu.VMEM((bpw, 64), jnp.bfloat16),
            pltpu.VMEM((bpw, 4, 108), jnp.int32),
            pltpu.VMEM((2 * _PAD_COLS, 64), jnp.bfloat16),
            pltpu.VMEM((_PAD_COLS,), jnp.float32),
            pltpu.SemaphoreType.DMA,
        ],
    )(iword, allidx3, wi, wo)


def _tc_loss_body(c, scores_ref, out_ref):
    x = scores_ref[...]
    rows, cols = x.shape
    col = lax.broadcasted_iota(jnp.int32, (rows, cols), 1)
    sign = jnp.where(col < c, 1.0, -1.0).astype(jnp.float32)
    z = x * sign
    ls = jnp.minimum(z, 0.0) - jnp.log1p(jnp.exp(-jnp.abs(z)))
    ls = jnp.where(col < c * (1 + _N_NEGS), ls, 0.0)
    out_ref[0, 0] = -jnp.sum(ls) / (rows * c)


def _tc_loss(scores, c):
    return pl.pallas_call(
        functools.partial(_tc_loss_body, c),
        out_shape=jax.ShapeDtypeStruct((1, 1), jnp.float32),
        out_specs=pl.BlockSpec(memory_space=pltpu.SMEM),
    )(scores)


def kernel(iword, owords, Wi, Wo):
    b = iword.shape[0]
    c = owords.shape[1]
    nwords = _neg_indices(b, c * _N_NEGS).astype(jnp.int32)
    pad = jnp.zeros((b, _PAD_COLS - c * (1 + _N_NEGS)), jnp.int32)
    allidx = jnp.concatenate(
        [owords.astype(jnp.int32), nwords, pad], axis=1)
    allidx3 = allidx.reshape(b, 4, 108)
    scores = _sc_scores(iword.astype(jnp.int32), allidx3,
                        Wi.astype(jnp.bfloat16), Wo.astype(jnp.bfloat16))
    loss = _tc_loss(scores, c)
    return loss.reshape(())


# speedup vs baseline: 8.4508x; 1.0000x over previous
import functools

import jax
import jax.numpy as jnp
import numpy as np
from jax import lax
from jax.experimental import pallas as pl
from jax.experimental.pallas import tpu as pltpu
from jax.experimental.pallas import tpu_sc as plsc

_N_NEGS = 20
_PAD_COLS = 432
_NCORES = 2
_NSUB = 16
_NW = _NCORES * _NSUB


def _neg_indices(b, n):
    nkey = jax.random.key(42)
    return jax.random.randint(nkey, (b, n), 0, 100000)


def _sc_scores_body(iword_hbm, idx_hbm, wi_hbm, wo_hbm, out_hbm,
                    iwd_v, ivec_v, idx_all_v, rows_v, score_v, sem):
    bpw = ivec_v.shape[0]
    c = lax.axis_index("c")
    s = lax.axis_index("s")
    wid = s * _NCORES + c
    base = wid * bpw

    pltpu.sync_copy(iword_hbm.at[pl.ds(base, bpw)], iwd_v)
    pltpu.sync_copy(idx_hbm.at[pl.ds(base, bpw)], idx_all_v)
    half = bpw // 2
    cp0 = pltpu.async_copy(wi_hbm.at[iwd_v.at[pl.ds(0, half)]],
                           ivec_v.at[pl.ds(0, half)], sem)
    cp1 = pltpu.async_copy(wi_hbm.at[iwd_v.at[pl.ds(half, half)]],
                           ivec_v.at[pl.ds(half, half)], sem)
    cp0.wait()
    cp1.wait()

    lane = lax.iota(jnp.int32, 16)

    def issue(bl, k):
        for j in range(4):
            pltpu.async_copy(
                wo_hbm.at[idx_all_v.at[bl, j]],
                rows_v.at[pl.ds(k * _PAD_COLS + j * 108, 108)], sem)

    def drain(bl, k):
        for j in range(4):
            pltpu.make_async_copy(
                wo_hbm.at[idx_all_v.at[bl, j]],
                rows_v.at[pl.ds(k * _PAD_COLS + j * 108, 108)], sem).wait()

    def pair_sum(p):
        u = plsc.bitcast(p, jnp.int32)
        hi = plsc.bitcast(u & jnp.int32(-65536), jnp.float32)
        lo = plsc.bitcast(u << 16, jnp.float32)
        return hi + lo

    def compute(bl, k):
        iv0 = ivec_v[bl, pl.ds(0, 32)]
        iv1 = ivec_v[bl, pl.ds(32, 32)]
        blm = bl % 32

        @plsc.parallel_loop(0, _PAD_COLS // 16, unroll=3)
        def g_body(g):
            acc = jnp.zeros((16,), jnp.float32)
            for rr in range(16):
                r = k * _PAD_COLS + g * 16 + rr
                p = (rows_v[r, pl.ds(0, 32)] * iv0
                     + rows_v[r, pl.ds(32, 32)] * iv1)
                sc = jnp.sum(pair_sum(p))
                acc = jnp.where(lane == rr, acc + sc, acc)
            score_v[blm, pl.ds(g * 16, 16)] = acc

        @pl.when(blm == 31)
        def _():
            pltpu.sync_copy(score_v,
                            out_hbm.at[pl.ds(base + bl - 31, 32)])

    issue(0, 0)

    def pair_body(bp, carry):
        b0 = 2 * bp
        b1 = b0 + 1
        issue(b1, 1)
        drain(b0, 0)
        compute(b0, 0)

        @pl.when(b0 + 2 < bpw)
        def _():
            issue(b0 + 2, 0)

        drain(b1, 1)
        compute(b1, 1)
        return carry

    lax.fori_loop(0, bpw // 2, pair_body, 0, unroll=False)


def _sc_scores(iword, allidx3, wi, wo):
    b = iword.shape[0]
    bpw = b // _NW
    mesh = plsc.VectorSubcoreMesh(core_axis_name="c", subcore_axis_name="s")
    return pl.kernel(
        _sc_scores_body,
        out_type=jax.ShapeDtypeStruct((b, _PAD_COLS), jnp.float32),
        mesh=mesh,
        compiler_params=pltpu.CompilerParams(
            needs_layout_passes=False, use_tc_tiling_on_sc=False),
        scratch_types=[
            pltpu.VMEM((bpw,), jnp.int32),
            pltpu.VMEM((bpw, 64), jnp.bfloat16),
            pltpu.VMEM((bpw, 4, 108), jnp.int32),
            pltpu.VMEM((2 * _PAD_COLS, 64), jnp.bfloat16),
            pltpu.VMEM((32, _PAD_COLS), jnp.float32),
            pltpu.SemaphoreType.DMA,
        ],
    )(iword, allidx3, wi, wo)


def _tc_loss_body(c, scores_ref, out_ref):
    x = scores_ref[...]
    rows, cols = x.shape
    col = lax.broadcasted_iota(jnp.int32, (rows, cols), 1)
    sign = jnp.where(col < c, 1.0, -1.0).astype(jnp.float32)
    z = x * sign
    ls = jnp.minimum(z, 0.0) - jnp.log1p(jnp.exp(-jnp.abs(z)))
    ls = jnp.where(col < c * (1 + _N_NEGS), ls, 0.0)
    out_ref[0, 0] = -jnp.sum(ls) / (rows * c)


def _tc_loss(scores, c):
    return pl.pallas_call(
        functools.partial(_tc_loss_body, c),
        out_shape=jax.ShapeDtypeStruct((1, 1), jnp.float32),
        out_specs=pl.BlockSpec(memory_space=pltpu.SMEM),
    )(scores)


def kernel(iword, owords, Wi, Wo):
    b = iword.shape[0]
    c = owords.shape[1]
    nwords = _neg_indices(b, c * _N_NEGS).astype(jnp.int32)
    pad = jnp.zeros((b, _PAD_COLS - c * (1 + _N_NEGS)), jnp.int32)
    allidx = jnp.concatenate(
        [owords.astype(jnp.int32), nwords, pad], axis=1)
    allidx3 = allidx.reshape(b, 4, 108)
    scores = _sc_scores(iword.astype(jnp.int32), allidx3,
                        Wi.astype(jnp.bfloat16), Wo.astype(jnp.bfloat16))
    loss = _tc_loss(scores, c)
    return loss.reshape(())
